# baseline (speedup 1.0000x reference)
import functools

import jax
import jax.numpy as jnp
from jax import lax
from jax.experimental import pallas as pl
from jax.experimental.pallas import tpu as pltpu
from jax.experimental.pallas import tpu_sc as plsc

NC = 2
NS = 16
LANES = 16
ECHUNK = 128
NH = 2
NBUF = 8
NR = 2


def _sc_mesh():
  return plsc.VectorSubcoreMesh(
      core_axis_name="c", subcore_axis_name="s", num_cores=NC,
      num_subcores=NS)


def _fill_zero_rows(buf, n_rows, width, dtype=jnp.float32):
  vw = LANES * (2 if dtype == jnp.bfloat16 else 1)
  def body(i, _):
    for k in range(width // vw):
      buf[i, pl.ds(k * vw, vw)] = jnp.zeros((vw,), dtype)
    return 0
  lax.fori_loop(0, n_rows, body, 0)


def _deg_body(n_nodes, nchunks, acc_rows, dst_hbm, deg_out, idx_v, ones_v,
              deg_sp):
  s = lax.axis_index("s")
  c = lax.axis_index("c")
  zero_per_tile = acc_rows // NS

  _fill_zero_rows(ones_v, ECHUNK, LANES)
  for k in range(zero_per_tile // ECHUNK):
    pltpu.sync_copy(ones_v,
                    deg_sp.at[pl.ds(s * zero_per_tile + k * ECHUNK, ECHUNK)])

  def fill_ones(i, _):
    ones_v[i] = jnp.ones((LANES,), jnp.float32)
    return 0
  lax.fori_loop(0, ECHUNK, fill_ones, 0)

  pltpu.sync_copy(dst_hbm.at[c, s], idx_v)
  plsc.subcore_barrier()

  def chunk(j, _):
    pltpu.sync_copy(ones_v, deg_sp.at[idx_v.at[j]], add=True)
    return 0
  lax.fori_loop(0, nchunks, chunk, 0)

  plsc.subcore_barrier()
  pltpu.sync_copy(deg_sp.at[pl.ds(s * zero_per_tile, zero_per_tile)],
                  deg_out.at[c, pl.ds(s * zero_per_tile, zero_per_tile)])


def _agg_body(nchunks, acc_rows, hw, gp_hbm, gn_hbm, srcp_hbm, dstp_hbm,
              srcn_hbm, dstn_hbm, acc_out, b0, b1, b2, b3, b4, b5, b6, b7,
              srcv, dstv, acc_sp, gsem, ssem):
  s = lax.axis_index("s")
  c = lax.axis_index("c")
  zero_per_tile = acc_rows // NS
  bufs = (b0, b1, b2, b3, b4, b5, b6, b7)

  rchunks = nchunks // NR
  for v in range(2):
    src_hbm = (srcp_hbm, srcn_hbm)[v]
    dst_hbm = (dstp_hbm, dstn_hbm)[v]
    table = ((gp_hbm, gn_hbm)[v]).at[c]

    pltpu.sync_copy(src_hbm.at[s], srcv)
    pltpu.sync_copy(dst_hbm.at[s], dstv)

    for r in range(NR):
      ro = r * rchunks

      _fill_zero_rows(b0, ECHUNK, hw, jnp.bfloat16)
      for k in range(zero_per_tile // ECHUNK):
        pltpu.sync_copy(
            b0, acc_sp.at[pl.ds(s * zero_per_tile + k * ECHUNK, ECHUNK)])
      plsc.subcore_barrier()

      for b in range(NBUF):
        pltpu.async_copy(table.at[srcv.at[ro + b]], bufs[b], gsem.at[b])

      def step(jj, _):
        base = ro + jj * NBUF
        for b in range(NBUF):
          j = base + b
          pltpu.make_async_copy(table.at[srcv.at[j]], bufs[b],
                                gsem.at[b]).wait()
          pltpu.async_copy(bufs[b], acc_sp.at[dstv.at[j]], ssem.at[b],
                           add=True)
        for b in range(NBUF):
          j = base + b

          @pl.when(j + NBUF < ro + rchunks)
          def _():
            pltpu.make_async_copy(bufs[b], acc_sp.at[dstv.at[j]],
                                  ssem.at[b]).wait()
            pltpu.async_copy(table.at[srcv.at[j + NBUF]], bufs[b], gsem.at[b])
        return 0
      lax.fori_loop(0, rchunks // NBUF, step, 0)

      for b in range(NBUF):
        j = ro + rchunks - NBUF + b
        pltpu.make_async_copy(bufs[b], acc_sp.at[dstv.at[j]],
                              ssem.at[b]).wait()

      plsc.subcore_barrier()
      pltpu.sync_copy(
          acc_sp.at[pl.ds(s * zero_per_tile, zero_per_tile)],
          acc_out.at[v, r, c, pl.ds(s * zero_per_tile, zero_per_tile)])


def _mm_body(x_ref, w_ref, deg_ref, g_ref):
  h = jnp.dot(x_ref[...], w_ref[0], preferred_element_type=jnp.float32)
  dinv = lax.rsqrt(deg_ref[...][:, 0:1] + 1.0)
  g_ref[...] = (h * dinv)[None].astype(jnp.bfloat16)


def _fin_body(acc_ref, gp_ref, gn_ref, dp_ref, dn_ref, bp_ref, bn_ref,
              o_ref):
  dinvp = lax.rsqrt(dp_ref[...][:, 0:1] + 1.0)
  dinvn = lax.rsqrt(dn_ref[...][:, 0:1] + 1.0)
  cols = []
  for h in range(NH):
    ap = gp_ref[h].astype(jnp.float32)
    an = gn_ref[h].astype(jnp.float32)
    for r in range(NR):
      ap = ap + acc_ref[0, r, h].astype(jnp.float32)
      an = an + acc_ref[1, r, h].astype(jnp.float32)
    zp = jnp.maximum(dinvp * ap + bp_ref[h][None], 0.0)
    zn = jnp.maximum(dinvn * an + bn_ref[h][None], 0.0)
    cols.append(zp - zn)
  o_ref[...] = jnp.concatenate(cols, axis=1)


def kernel(x, edge_index_pos, edge_index_neg, W_pos, b_pos, W_neg, b_neg):
  n_nodes, d_in = x.shape
  d_out = W_pos.shape[1]
  hw = d_out // NH
  n_edges = edge_index_pos.shape[1]

  nchunks = NBUF * ((n_edges + (NS * ECHUNK * NBUF) - 1) //
                    (NS * ECHUNK * NBUF))
  e_pad = NS * nchunks * ECHUNK
  acc_rows = ((n_nodes + 1 + NS * ECHUNK - 1) // (NS * ECHUNK)) * NS * ECHUNK
  dummy = n_nodes

  def prep(ei):
    src = ei[0].astype(jnp.int32)
    dst = ei[1].astype(jnp.int32)
    pad = e_pad - n_edges
    src = jnp.concatenate([src, jnp.zeros((pad,), jnp.int32)])
    dst = jnp.concatenate([dst, jnp.full((pad,), dummy, jnp.int32)])
    return (src.reshape(NS, nchunks, ECHUNK),
            dst.reshape(NS, nchunks, ECHUNK))

  src_p, dst_p = prep(edge_index_pos)
  src_n, dst_n = prep(edge_index_neg)

  mesh = _sc_mesh()
  sc_params = pltpu.CompilerParams(use_tc_tiling_on_sc=False,
                                   internal_scratch_in_bytes=1 << 16)

  deg16 = pl.kernel(
      functools.partial(_deg_body, n_nodes, nchunks, acc_rows),
      out_type=jax.ShapeDtypeStruct((NC, acc_rows, LANES), jnp.float32),
      mesh=mesh,
      compiler_params=sc_params,
      scratch_types=[
          pltpu.VMEM((nchunks, ECHUNK), jnp.int32),
          pltpu.VMEM((ECHUNK, LANES), jnp.float32),
          pltpu.VMEM_SHARED((acc_rows, LANES), jnp.float32),
      ],
  )(jnp.stack([dst_p, dst_n]))

  rblk = 1000
  ngrid = n_nodes // rblk

  def matmul(w, deg):
    return pl.pallas_call(
        _mm_body,
        grid=(ngrid, NH),
        in_specs=[
            pl.BlockSpec((rblk, d_in), lambda r, h: (r, 0)),
            pl.BlockSpec((1, d_in, hw), lambda r, h: (h, 0, 0)),
            pl.BlockSpec((rblk, LANES), lambda r, h: (r, 0)),
        ],
        out_specs=pl.BlockSpec((1, rblk, hw), lambda r, h: (h, r, 0)),
        out_shape=jax.ShapeDtypeStruct((NH, n_nodes, hw), jnp.bfloat16),
    )(x, w.reshape(d_in, NH, hw).transpose(1, 0, 2), deg)

  g_p = matmul(W_pos, deg16[0])
  g_n = matmul(W_neg, deg16[1])

  agg = pl.kernel(
      functools.partial(_agg_body, nchunks, acc_rows, hw),
      out_type=jax.ShapeDtypeStruct((2, NR, NH, acc_rows, hw), jnp.bfloat16),
      mesh=mesh,
      compiler_params=sc_params,
      scratch_types=[
          pltpu.VMEM((ECHUNK, hw), jnp.bfloat16),
          pltpu.VMEM((ECHUNK, hw), jnp.bfloat16),
          pltpu.VMEM((ECHUNK, hw), jnp.bfloat16),
          pltpu.VMEM((ECHUNK, hw), jnp.bfloat16),
          pltpu.VMEM((ECHUNK, hw), jnp.bfloat16),
          pltpu.VMEM((ECHUNK, hw), jnp.bfloat16),
          pltpu.VMEM((ECHUNK, hw), jnp.bfloat16),
          pltpu.VMEM((ECHUNK, hw), jnp.bfloat16),
          pltpu.VMEM((nchunks, ECHUNK), jnp.int32),
          pltpu.VMEM((nchunks, ECHUNK), jnp.int32),
          pltpu.VMEM_SHARED((acc_rows, hw), jnp.bfloat16),
          pltpu.SemaphoreType.DMA((NBUF,)),
          pltpu.SemaphoreType.DMA((NBUF,)),
      ],
  )

  acc2 = agg(g_p, g_n, src_p, dst_p, src_n, dst_n)

  out = pl.pallas_call(
      _fin_body,
      grid=(ngrid,),
      in_specs=[
          pl.BlockSpec((2, NR, NH, rblk, hw), lambda r: (0, 0, 0, r, 0)),
          pl.BlockSpec((NH, rblk, hw), lambda r: (0, r, 0)),
          pl.BlockSpec((NH, rblk, hw), lambda r: (0, r, 0)),
          pl.BlockSpec((rblk, LANES), lambda r: (r, 0)),
          pl.BlockSpec((rblk, LANES), lambda r: (r, 0)),
          pl.BlockSpec((NH, hw), lambda r: (0, 0)),
          pl.BlockSpec((NH, hw), lambda r: (0, 0)),
      ],
      out_specs=pl.BlockSpec((rblk, d_out), lambda r: (r, 0)),
      out_shape=jax.ShapeDtypeStruct((n_nodes, d_out), jnp.float32),
  )(acc2, g_p, g_n, deg16[0], deg16[1],
    b_pos.reshape(NH, hw), b_neg.reshape(NH, hw))

  return out


# speedup vs baseline: 1.1177x; 1.1177x over previous
import functools

import jax
import jax.numpy as jnp
from jax import lax
from jax.experimental import pallas as pl
from jax.experimental.pallas import tpu as pltpu
from jax.experimental.pallas import tpu_sc as plsc

NC = 2
NS = 16
LANES = 16
ECHUNK = 128
NH = 2
NBUF = 8
NR = 1


def _sc_mesh():
  return plsc.VectorSubcoreMesh(
      core_axis_name="c", subcore_axis_name="s", num_cores=NC,
      num_subcores=NS)


def _fill_zero_rows(buf, n_rows, width, dtype=jnp.float32):
  vw = LANES * (2 if dtype == jnp.bfloat16 else 1)
  def body(i, _):
    for k in range(width // vw):
      buf[i, pl.ds(k * vw, vw)] = jnp.zeros((vw,), dtype)
    return 0
  lax.fori_loop(0, n_rows, body, 0)


def _deg_body(n_nodes, nchunks, acc_rows, dst_hbm, deg_out, idx_v, ones_v,
              deg_sp):
  s = lax.axis_index("s")
  c = lax.axis_index("c")
  zero_per_tile = acc_rows // NS

  _fill_zero_rows(ones_v, ECHUNK, LANES)
  for k in range(zero_per_tile // ECHUNK):
    pltpu.sync_copy(ones_v,
                    deg_sp.at[pl.ds(s * zero_per_tile + k * ECHUNK, ECHUNK)])

  def fill_ones(i, _):
    ones_v[i] = jnp.ones((LANES,), jnp.float32)
    return 0
  lax.fori_loop(0, ECHUNK, fill_ones, 0)

  pltpu.sync_copy(dst_hbm.at[c, s], idx_v)
  plsc.subcore_barrier()

  def chunk(j, _):
    pltpu.sync_copy(ones_v, deg_sp.at[idx_v.at[j]], add=True)
    return 0
  lax.fori_loop(0, nchunks, chunk, 0)

  plsc.subcore_barrier()
  pltpu.sync_copy(deg_sp.at[pl.ds(s * zero_per_tile, zero_per_tile)],
                  deg_out.at[c, pl.ds(s * zero_per_tile, zero_per_tile)])


def _agg_body(nchunks, acc_rows, hw, g2_hbm, srcp_hbm, dstp_hbm,
              srcn_hbm, dstn_hbm, acc_out, b0, b1, b2, b3, b4, b5, b6, b7,
              srcv, dstv, acc_sp, gsem, ssem):
  s = lax.axis_index("s")
  c = lax.axis_index("c")
  zero_per_tile = acc_rows // NS
  bufs = (b0, b1, b2, b3, b4, b5, b6, b7)

  rchunks = nchunks // NR
  for v in range(2):
    src_hbm = (srcp_hbm, srcn_hbm)[v]
    dst_hbm = (dstp_hbm, dstn_hbm)[v]
    table = g2_hbm.at[v, c]

    pltpu.sync_copy(src_hbm.at[s], srcv)
    pltpu.sync_copy(dst_hbm.at[s], dstv)

    for r in range(NR):
      ro = r * rchunks

      _fill_zero_rows(b0, ECHUNK, hw, jnp.bfloat16)
      for k in range(zero_per_tile // ECHUNK):
        pltpu.sync_copy(
            b0, acc_sp.at[pl.ds(s * zero_per_tile + k * ECHUNK, ECHUNK)])
      plsc.subcore_barrier()

      for b in range(NBUF):
        pltpu.async_copy(table.at[srcv.at[ro + b]], bufs[b], gsem.at[b])

      def step(jj, _):
        base = ro + jj * NBUF
        for b in range(NBUF):
          j = base + b
          pltpu.make_async_copy(table.at[srcv.at[j]], bufs[b],
                                gsem.at[b]).wait()
          pltpu.async_copy(bufs[b], acc_sp.at[dstv.at[j]], ssem.at[b],
                           add=True)
        for b in range(NBUF):
          j = base + b

          @pl.when(j + NBUF < ro + rchunks)
          def _():
            pltpu.make_async_copy(bufs[b], acc_sp.at[dstv.at[j]],
                                  ssem.at[b]).wait()
            pltpu.async_copy(table.at[srcv.at[j + NBUF]], bufs[b], gsem.at[b])
        return 0
      lax.fori_loop(0, rchunks // NBUF, step, 0)

      for b in range(NBUF):
        j = ro + rchunks - NBUF + b
        pltpu.make_async_copy(bufs[b], acc_sp.at[dstv.at[j]],
                              ssem.at[b]).wait()

      plsc.subcore_barrier()
      pltpu.sync_copy(
          acc_sp.at[pl.ds(s * zero_per_tile, zero_per_tile)],
          acc_out.at[v, r, c, pl.ds(s * zero_per_tile, zero_per_tile)])


def _mm_body(x_ref, w_ref, deg_ref, g_ref):
  h = jnp.dot(x_ref[...], w_ref[0, 0], preferred_element_type=jnp.float32)
  dinv = lax.rsqrt(deg_ref[0][:, 0:1] + 1.0)
  g_ref[...] = (h * dinv)[None, None].astype(jnp.bfloat16)


def _fin_body(acc_ref, g_ref, dp_ref, dn_ref, bp_ref, bn_ref, o_ref):
  dinvp = lax.rsqrt(dp_ref[...][:, 0:1] + 1.0)
  dinvn = lax.rsqrt(dn_ref[...][:, 0:1] + 1.0)
  cols = []
  for h in range(NH):
    ap = g_ref[0, h].astype(jnp.float32)
    an = g_ref[1, h].astype(jnp.float32)
    for r in range(NR):
      ap = ap + acc_ref[0, r, h].astype(jnp.float32)
      an = an + acc_ref[1, r, h].astype(jnp.float32)
    zp = jnp.maximum(dinvp * ap + bp_ref[h][None], 0.0)
    zn = jnp.maximum(dinvn * an + bn_ref[h][None], 0.0)
    cols.append(zp - zn)
  o_ref[...] = jnp.concatenate(cols, axis=1)


def kernel(x, edge_index_pos, edge_index_neg, W_pos, b_pos, W_neg, b_neg):
  n_nodes, d_in = x.shape
  d_out = W_pos.shape[1]
  hw = d_out // NH
  n_edges = edge_index_pos.shape[1]

  nchunks = NBUF * ((n_edges + (NS * ECHUNK * NBUF) - 1) //
                    (NS * ECHUNK * NBUF))
  e_pad = NS * nchunks * ECHUNK
  acc_rows = ((n_nodes + 1 + NS * ECHUNK - 1) // (NS * ECHUNK)) * NS * ECHUNK
  dummy = n_nodes

  def prep(ei):
    src = ei[0].astype(jnp.int32)
    dst = ei[1].astype(jnp.int32)
    pad = e_pad - n_edges
    src = jnp.concatenate([src, jnp.zeros((pad,), jnp.int32)])
    dst = jnp.concatenate([dst, jnp.full((pad,), dummy, jnp.int32)])
    return (src.reshape(NS, nchunks, ECHUNK),
            dst.reshape(NS, nchunks, ECHUNK))

  src_p, dst_p = prep(edge_index_pos)
  src_n, dst_n = prep(edge_index_neg)

  mesh = _sc_mesh()
  sc_params = pltpu.CompilerParams(use_tc_tiling_on_sc=False,
                                   internal_scratch_in_bytes=1 << 16)

  deg16 = pl.kernel(
      functools.partial(_deg_body, n_nodes, nchunks, acc_rows),
      out_type=jax.ShapeDtypeStruct((NC, acc_rows, LANES), jnp.float32),
      mesh=mesh,
      compiler_params=sc_params,
      scratch_types=[
          pltpu.VMEM((nchunks, ECHUNK), jnp.int32),
          pltpu.VMEM((ECHUNK, LANES), jnp.float32),
          pltpu.VMEM_SHARED((acc_rows, LANES), jnp.float32),
      ],
  )(jnp.stack([dst_p, dst_n]))

  rblk = 1000
  ngrid = n_nodes // rblk

  w2 = jnp.stack([
      W_pos.reshape(d_in, NH, hw).transpose(1, 0, 2),
      W_neg.reshape(d_in, NH, hw).transpose(1, 0, 2),
  ])
  g2 = pl.pallas_call(
      _mm_body,
      grid=(ngrid, 2, NH),
      in_specs=[
          pl.BlockSpec((rblk, d_in), lambda r, v, h: (r, 0)),
          pl.BlockSpec((1, 1, d_in, hw), lambda r, v, h: (v, h, 0, 0)),
          pl.BlockSpec((1, rblk, LANES), lambda r, v, h: (v, r, 0)),
      ],
      out_specs=pl.BlockSpec((1, 1, rblk, hw), lambda r, v, h: (v, h, r, 0)),
      out_shape=jax.ShapeDtypeStruct((2, NH, n_nodes, hw), jnp.bfloat16),
  )(x, w2, deg16)

  agg = pl.kernel(
      functools.partial(_agg_body, nchunks, acc_rows, hw),
      out_type=jax.ShapeDtypeStruct((2, NR, NH, acc_rows, hw), jnp.bfloat16),
      mesh=mesh,
      compiler_params=sc_params,
      scratch_types=[
          pltpu.VMEM((ECHUNK, hw), jnp.bfloat16),
          pltpu.VMEM((ECHUNK, hw), jnp.bfloat16),
          pltpu.VMEM((ECHUNK, hw), jnp.bfloat16),
          pltpu.VMEM((ECHUNK, hw), jnp.bfloat16),
          pltpu.VMEM((ECHUNK, hw), jnp.bfloat16),
          pltpu.VMEM((ECHUNK, hw), jnp.bfloat16),
          pltpu.VMEM((ECHUNK, hw), jnp.bfloat16),
          pltpu.VMEM((ECHUNK, hw), jnp.bfloat16),
          pltpu.VMEM((nchunks, ECHUNK), jnp.int32),
          pltpu.VMEM((nchunks, ECHUNK), jnp.int32),
          pltpu.VMEM_SHARED((acc_rows, hw), jnp.bfloat16),
          pltpu.SemaphoreType.DMA((NBUF,)),
          pltpu.SemaphoreType.DMA((NBUF,)),
      ],
  )

  acc2 = agg(g2, src_p, dst_p, src_n, dst_n)

  out = pl.pallas_call(
      _fin_body,
      grid=(ngrid,),
      in_specs=[
          pl.BlockSpec((2, NR, NH, rblk, hw), lambda r: (0, 0, 0, r, 0)),
          pl.BlockSpec((2, NH, rblk, hw), lambda r: (0, 0, r, 0)),
          pl.BlockSpec((rblk, LANES), lambda r: (r, 0)),
          pl.BlockSpec((rblk, LANES), lambda r: (r, 0)),
          pl.BlockSpec((NH, hw), lambda r: (0, 0)),
          pl.BlockSpec((NH, hw), lambda r: (0, 0)),
      ],
      out_specs=pl.BlockSpec((rblk, d_out), lambda r: (r, 0)),
      out_shape=jax.ShapeDtypeStruct((n_nodes, d_out), jnp.float32),
  )(acc2, g2, deg16[0], deg16[1],
    b_pos.reshape(NH, hw), b_neg.reshape(NH, hw))

  return out


# speedup vs baseline: 1.2609x; 1.1282x over previous
import functools

import jax
import jax.numpy as jnp
from jax import lax
from jax.experimental import pallas as pl
from jax.experimental.pallas import tpu as pltpu
from jax.experimental.pallas import tpu_sc as plsc

NC = 2
NS = 16
LANES = 16
ECHUNK = 128
NQ = 4
NBUF = 8


def _sc_mesh():
  return plsc.VectorSubcoreMesh(
      core_axis_name="c", subcore_axis_name="s", num_cores=NC,
      num_subcores=NS)


def _fill_zero_rows(buf, n_rows, width, dtype=jnp.float32):
  vw = LANES * (2 if dtype == jnp.bfloat16 else 1)
  def body(i, _):
    for k in range(width // vw):
      buf[i, pl.ds(k * vw, vw)] = jnp.zeros((vw,), dtype)
    return 0
  lax.fori_loop(0, n_rows, body, 0)


def _deg_body(n_nodes, nchunks, acc_rows, dst_hbm, deg_out, idx_v, ones_v,
              deg_sp):
  s = lax.axis_index("s")
  c = lax.axis_index("c")
  zero_per_tile = acc_rows // NS

  _fill_zero_rows(ones_v, ECHUNK, LANES)
  for k in range(zero_per_tile // ECHUNK):
    pltpu.sync_copy(ones_v,
                    deg_sp.at[pl.ds(s * zero_per_tile + k * ECHUNK, ECHUNK)])

  def fill_ones(i, _):
    ones_v[i] = jnp.ones((LANES,), jnp.float32)
    return 0
  lax.fori_loop(0, ECHUNK, fill_ones, 0)

  pltpu.sync_copy(dst_hbm.at[c, s], idx_v)
  plsc.subcore_barrier()

  def chunk(j, _):
    pltpu.sync_copy(ones_v, deg_sp.at[idx_v.at[j]], add=True)
    return 0
  lax.fori_loop(0, nchunks, chunk, 0)

  plsc.subcore_barrier()
  pltpu.sync_copy(deg_sp.at[pl.ds(s * zero_per_tile, zero_per_tile)],
                  deg_out.at[c, pl.ds(s * zero_per_tile, zero_per_tile)])


def _agg_body(nchunks, acc_rows, qw, g2_hbm, srcp_hbm, dstp_hbm,
              srcn_hbm, dstn_hbm, acc_out, b0, b1, b2, b3, b4, b5, b6, b7,
              srcv, dstv, table_sp, acc_sp, gsem, ssem):
  s = lax.axis_index("s")
  c = lax.axis_index("c")
  zero_per_tile = acc_rows // NS
  bufs = (b0, b1, b2, b3, b4, b5, b6, b7)

  for v in range(2):
    src_hbm = (srcp_hbm, srcn_hbm)[v]
    dst_hbm = (dstp_hbm, dstn_hbm)[v]

    pltpu.sync_copy(src_hbm.at[s], srcv)
    pltpu.sync_copy(dst_hbm.at[s], dstv)

    for ql in range(NQ // NC):
      q = c * (NQ // NC) + ql
      tq = g2_hbm.at[v, q]

      for k in range(zero_per_tile // ECHUNK):
        ro = s * zero_per_tile + k * ECHUNK
        pltpu.sync_copy(tq.at[pl.ds(ro, ECHUNK)], b0)
        pltpu.sync_copy(b0, table_sp.at[pl.ds(ro, ECHUNK)])

      _fill_zero_rows(b0, ECHUNK, qw, jnp.bfloat16)
      for k in range(zero_per_tile // ECHUNK):
        pltpu.sync_copy(
            b0, acc_sp.at[pl.ds(s * zero_per_tile + k * ECHUNK, ECHUNK)])
      plsc.subcore_barrier()

      for b in range(NBUF):
        pltpu.async_copy(table_sp.at[srcv.at[b]], bufs[b], gsem.at[b])

      def step(jj, _):
        base = jj * NBUF
        for b in range(NBUF):
          j = base + b
          pltpu.make_async_copy(table_sp.at[srcv.at[j]], bufs[b],
                                gsem.at[b]).wait()
          pltpu.async_copy(bufs[b], acc_sp.at[dstv.at[j]], ssem.at[b],
                           add=True)
        for b in range(NBUF):
          j = base + b

          @pl.when(j + NBUF < nchunks)
          def _():
            pltpu.make_async_copy(bufs[b], acc_sp.at[dstv.at[j]],
                                  ssem.at[b]).wait()
            pltpu.async_copy(table_sp.at[srcv.at[j + NBUF]], bufs[b],
                             gsem.at[b])
        return 0
      lax.fori_loop(0, nchunks // NBUF, step, 0)

      for b in range(NBUF):
        j = nchunks - NBUF + b
        pltpu.make_async_copy(bufs[b], acc_sp.at[dstv.at[j]],
                              ssem.at[b]).wait()

      plsc.subcore_barrier()
      pltpu.sync_copy(
          acc_sp.at[pl.ds(s * zero_per_tile, zero_per_tile)],
          acc_out.at[v, q, pl.ds(s * zero_per_tile, zero_per_tile)])


def _mm_body(x_ref, w_ref, deg_ref, g_ref):
  h = jnp.dot(x_ref[...], w_ref[0, 0], preferred_element_type=jnp.float32)
  dinv = lax.rsqrt(deg_ref[0][:, 0:1] + 1.0)
  g_ref[...] = (h * dinv)[None, None].astype(jnp.bfloat16)


def _fin_body(acc_ref, g_ref, dp_ref, dn_ref, bp_ref, bn_ref, o_ref):
  dinvp = lax.rsqrt(dp_ref[...][:, 0:1] + 1.0)
  dinvn = lax.rsqrt(dn_ref[...][:, 0:1] + 1.0)
  cols = []
  for q in range(NQ):
    ap = g_ref[0, q].astype(jnp.float32) + acc_ref[0, q].astype(jnp.float32)
    an = g_ref[1, q].astype(jnp.float32) + acc_ref[1, q].astype(jnp.float32)
    zp = jnp.maximum(dinvp * ap + bp_ref[q][None], 0.0)
    zn = jnp.maximum(dinvn * an + bn_ref[q][None], 0.0)
    cols.append(zp - zn)
  o_ref[...] = jnp.concatenate(cols, axis=1)


def kernel(x, edge_index_pos, edge_index_neg, W_pos, b_pos, W_neg, b_neg):
  n_nodes, d_in = x.shape
  d_out = W_pos.shape[1]
  qw = d_out // NQ
  n_edges = edge_index_pos.shape[1]

  nchunks = NBUF * ((n_edges + (NS * ECHUNK * NBUF) - 1) //
                    (NS * ECHUNK * NBUF))
  e_pad = NS * nchunks * ECHUNK
  acc_rows = ((n_nodes + 1 + NS * ECHUNK - 1) // (NS * ECHUNK)) * NS * ECHUNK
  dummy = n_nodes

  def prep(ei):
    src = ei[0].astype(jnp.int32)
    dst = ei[1].astype(jnp.int32)
    pad = e_pad - n_edges
    src = jnp.concatenate([src, jnp.zeros((pad,), jnp.int32)])
    dst = jnp.concatenate([dst, jnp.full((pad,), dummy, jnp.int32)])
    return (src.reshape(NS, nchunks, ECHUNK),
            dst.reshape(NS, nchunks, ECHUNK))

  src_p, dst_p = prep(edge_index_pos)
  src_n, dst_n = prep(edge_index_neg)

  mesh = _sc_mesh()
  sc_params = pltpu.CompilerParams(use_tc_tiling_on_sc=False,
                                   internal_scratch_in_bytes=1 << 16)

  deg16 = pl.kernel(
      functools.partial(_deg_body, n_nodes, nchunks, acc_rows),
      out_type=jax.ShapeDtypeStruct((NC, acc_rows, LANES), jnp.float32),
      mesh=mesh,
      compiler_params=sc_params,
      scratch_types=[
          pltpu.VMEM((nchunks, ECHUNK), jnp.int32),
          pltpu.VMEM((ECHUNK, LANES), jnp.float32),
          pltpu.VMEM_SHARED((acc_rows, LANES), jnp.float32),
      ],
  )(jnp.stack([dst_p, dst_n]))

  rblk = 1000
  ngrid = n_nodes // rblk

  w2 = jnp.stack([
      W_pos.reshape(d_in, NQ, qw).transpose(1, 0, 2),
      W_neg.reshape(d_in, NQ, qw).transpose(1, 0, 2),
  ])
  g2 = pl.pallas_call(
      _mm_body,
      grid=(ngrid, 2, NQ),
      in_specs=[
          pl.BlockSpec((rblk, d_in), lambda r, v, q: (r, 0)),
          pl.BlockSpec((1, 1, d_in, qw), lambda r, v, q: (v, q, 0, 0)),
          pl.BlockSpec((1, rblk, LANES), lambda r, v, q: (v, r, 0)),
      ],
      out_specs=pl.BlockSpec((1, 1, rblk, qw), lambda r, v, q: (v, q, r, 0)),
      out_shape=jax.ShapeDtypeStruct((2, NQ, acc_rows, qw), jnp.bfloat16),
  )(x, w2, deg16)

  agg = pl.kernel(
      functools.partial(_agg_body, nchunks, acc_rows, qw),
      out_type=jax.ShapeDtypeStruct((2, NQ, acc_rows, qw), jnp.bfloat16),
      mesh=mesh,
      compiler_params=sc_params,
      scratch_types=[
          pltpu.VMEM((ECHUNK, qw), jnp.bfloat16),
          pltpu.VMEM((ECHUNK, qw), jnp.bfloat16),
          pltpu.VMEM((ECHUNK, qw), jnp.bfloat16),
          pltpu.VMEM((ECHUNK, qw), jnp.bfloat16),
          pltpu.VMEM((ECHUNK, qw), jnp.bfloat16),
          pltpu.VMEM((ECHUNK, qw), jnp.bfloat16),
          pltpu.VMEM((ECHUNK, qw), jnp.bfloat16),
          pltpu.VMEM((ECHUNK, qw), jnp.bfloat16),
          pltpu.VMEM((nchunks, ECHUNK), jnp.int32),
          pltpu.VMEM((nchunks, ECHUNK), jnp.int32),
          pltpu.VMEM_SHARED((acc_rows, qw), jnp.bfloat16),
          pltpu.VMEM_SHARED((acc_rows, qw), jnp.bfloat16),
          pltpu.SemaphoreType.DMA((NBUF,)),
          pltpu.SemaphoreType.DMA((NBUF,)),
      ],
  )

  acc2 = agg(g2, src_p, dst_p, src_n, dst_n)

  out = pl.pallas_call(
      _fin_body,
      grid=(ngrid,),
      in_specs=[
          pl.BlockSpec((2, NQ, rblk, qw), lambda r: (0, 0, r, 0)),
          pl.BlockSpec((2, NQ, rblk, qw), lambda r: (0, 0, r, 0)),
          pl.BlockSpec((rblk, LANES), lambda r: (r, 0)),
          pl.BlockSpec((rblk, LANES), lambda r: (r, 0)),
          pl.BlockSpec((NQ, qw), lambda r: (0, 0)),
          pl.BlockSpec((NQ, qw), lambda r: (0, 0)),
      ],
      out_specs=pl.BlockSpec((rblk, d_out), lambda r: (r, 0)),
      out_shape=jax.ShapeDtypeStruct((n_nodes, d_out), jnp.float32),
  )(acc2, g2, deg16[0], deg16[1],
    b_pos.reshape(NQ, qw), b_neg.reshape(NQ, qw))

  return out
